# Initial kernel scaffold; baseline (speedup 1.0000x reference)
import jax
import jax.numpy as jnp
from jax.experimental import pallas as pl


def kernel(reg_info, inputs, edge_index, W0, a_src0, a_dst0, W1, a_src1, a_dst1):
    raise NotImplementedError("write your pallas kernel here")


# speedup vs baseline: 1.0917x; 1.0917x over previous
import functools

import jax
import jax.numpy as jnp
from jax import lax
from jax.experimental import pallas as pl
from jax.experimental.pallas import tpu as pltpu

_PREC = lax.Precision.HIGHEST


def _layer_tc(x, W, as_flat, ad_flat, apply_elu):
    N, D = x.shape
    K = W.shape[1]
    H = as_flat.shape[1]
    BM = 1000
    grid = N // BM

    def body(x_ref, w_ref, asf_ref, adf_ref, h_ref, as_ref, ad_ref, mx_ref):
        xb = x_ref[...]
        if apply_elu:
            xb = jnp.where(xb > 0, xb, jnp.exp(jnp.minimum(xb, 0.0)) - 1.0)
        h = lax.dot_general(xb, w_ref[...], (((1,), (0,)), ((), ())),
                            precision=_PREC)
        h_ref[...] = h
        asb = lax.dot_general(h, asf_ref[...], (((1,), (0,)), ((), ())),
                              precision=_PREC)
        adb = lax.dot_general(h, adf_ref[...], (((1,), (0,)), ((), ())),
                              precision=_PREC)
        as_ref[...] = asb
        ad_ref[...] = adb
        mx_ref[...] = jnp.stack([jnp.max(asb), jnp.max(adb)]).reshape(1, 1, 2)

    return pl.pallas_call(
        body,
        grid=(grid,),
        in_specs=[pl.BlockSpec((BM, D), lambda i: (i, 0)),
                  pl.BlockSpec((D, K), lambda i: (0, 0)),
                  pl.BlockSpec((K, H), lambda i: (0, 0)),
                  pl.BlockSpec((K, H), lambda i: (0, 0))],
        out_specs=[pl.BlockSpec((BM, K), lambda i: (i, 0)),
                   pl.BlockSpec((BM, H), lambda i: (i, 0)),
                   pl.BlockSpec((BM, H), lambda i: (i, 0)),
                   pl.BlockSpec((1, 1, 2), lambda i: (i, 0, 0))],
        out_shape=[jax.ShapeDtypeStruct((N, K), jnp.float32),
                   jax.ShapeDtypeStruct((N, H), jnp.float32),
                   jax.ShapeDtypeStruct((N, H), jnp.float32),
                   jax.ShapeDtypeStruct((grid, 1, 2), jnp.float32)],
    )(x, W, as_flat, ad_flat)


def _expand_head_weights(a, heads, out_dim):
    z = jnp.zeros((heads, out_dim, heads), dtype=a.dtype)
    z = z.at[jnp.arange(heads), :, jnp.arange(heads)].set(a)
    return z.reshape(heads * out_dim, heads)


def _edge_phase(h, alpha_s, alpha_d, b, src, dst, heads, out_dim, n):
    e = alpha_s[src] + alpha_d[dst]
    e = jnp.where(e > 0, e, 0.2 * e)
    ex = jnp.exp(e - b)
    denom = jax.ops.segment_sum(ex, dst, num_segments=n)
    hr = h.reshape(n, heads, out_dim)
    msg = hr[src] * ex[:, :, None]
    agg = jax.ops.segment_sum(msg, dst, num_segments=n)
    out = agg / (denom[:, :, None] + 1e-16)
    alpha = ex / (denom[dst] + 1e-16)
    return out.reshape(n, heads * out_dim), alpha


def kernel(reg_info, inputs, edge_index, W0, a_src0, a_dst0, W1, a_src1,
           a_dst1):
    x = inputs[0]
    n = x.shape[0]
    d = x.shape[1]
    heads = a_src0.shape[0]
    src = edge_index[0]
    dst = edge_index[1]

    as0_flat = _expand_head_weights(a_src0, heads, d)
    ad0_flat = _expand_head_weights(a_dst0, heads, d)
    h0, as0, ad0, mx0 = _layer_tc(x, W0, as0_flat, ad0_flat, apply_elu=False)
    b0 = jnp.max(mx0[:, 0, 0]) + jnp.max(mx0[:, 0, 1])

    agg0, alpha0 = _edge_phase(h0, as0, ad0, b0, src, dst, heads, d, n)

    as1_flat = a_src1.reshape(d, 1)
    ad1_flat = a_dst1.reshape(d, 1)
    h1, as1, ad1, mx1 = _layer_tc(agg0, W1, as1_flat, ad1_flat,
                                  apply_elu=True)
    b1 = jnp.max(mx1[:, 0, 0]) + jnp.max(mx1[:, 0, 1])

    x2, _ = _edge_phase(h1, as1, ad1, b1, src, dst, 1, d, n)

    outputs = x2.reshape(1, n, d)
    alpha_norm = alpha0.mean(-1)
    return (outputs, alpha_norm)
